# baseline (speedup 1.0000x reference)
import jax
import jax.numpy as jnp
from jax.experimental import pallas as pl
from jax.experimental.pallas import tpu as pltpu


def _body(idx_ref, arr_ref, out_ref, sem):
    i = idx_ref[0]
    copy = pltpu.make_async_copy(arr_ref.at[i], out_ref, sem)
    copy.start()
    copy.wait()


def kernel(x, arr, index):
    del x
    t, r, d = arr.shape
    idx = jnp.reshape(jnp.asarray(index, jnp.int32), (1,))
    arr_t = jnp.transpose(arr, (0, 2, 1))
    out_t = pl.pallas_call(
        _body,
        out_shape=jax.ShapeDtypeStruct((d, r), jnp.float32),
        in_specs=[
            pl.BlockSpec(memory_space=pltpu.MemorySpace.SMEM),
            pl.BlockSpec(memory_space=pl.ANY),
        ],
        out_specs=pl.BlockSpec(memory_space=pl.ANY),
        scratch_shapes=[pltpu.SemaphoreType.DMA],
    )(idx, arr_t)
    return out_t.T


# speedup vs baseline: 8.1725x; 8.1725x over previous
import jax
import jax.numpy as jnp
from jax.experimental import pallas as pl
from jax.experimental.pallas import tpu as pltpu


def _body(idx_ref, arr_ref, out_ref):
    del idx_ref
    out_ref[...] = arr_ref[0]


def kernel(x, arr, index):
    del x
    t, r, d = arr.shape
    idx = jnp.reshape(jnp.asarray(index, jnp.int32), (1,))
    arr_t = jnp.transpose(arr, (0, 2, 1))
    nblk = 4
    grid_spec = pltpu.PrefetchScalarGridSpec(
        num_scalar_prefetch=1,
        grid=(nblk,),
        in_specs=[
            pl.BlockSpec((1, d, r // nblk), lambda i, idx_ref: (idx_ref[0], 0, i))
        ],
        out_specs=pl.BlockSpec((d, r // nblk), lambda i, idx_ref: (0, i)),
    )
    out_t = pl.pallas_call(
        _body,
        grid_spec=grid_spec,
        out_shape=jax.ShapeDtypeStruct((d, r), jnp.float32),
    )(idx, arr_t)
    return out_t.T


# speedup vs baseline: 11.6763x; 1.4287x over previous
import jax
import jax.numpy as jnp
from jax.experimental import pallas as pl
from jax.experimental.pallas import tpu as pltpu


def _body(idx_ref, arr_ref, out_ref, sem):
    i = idx_ref[0]
    copy = pltpu.make_async_copy(arr_ref.at[i], out_ref, sem)
    copy.start()
    copy.wait()


def kernel(x, arr, index):
    del x
    t, r, d = arr.shape
    idx = jnp.reshape(jnp.asarray(index, jnp.int32), (1,))
    arr_t = jnp.transpose(arr, (0, 2, 1))
    out_t = pl.pallas_call(
        _body,
        out_shape=jax.ShapeDtypeStruct((d, r), jnp.float32),
        in_specs=[
            pl.BlockSpec(memory_space=pltpu.MemorySpace.SMEM),
            pl.BlockSpec(memory_space=pl.ANY),
        ],
        out_specs=pl.BlockSpec((d, r), lambda: (0, 0)),
        scratch_shapes=[pltpu.SemaphoreType.DMA],
    )(idx, arr_t)
    return out_t.T


# speedup vs baseline: 12.1613x; 1.0415x over previous
import jax
import jax.numpy as jnp
from jax.experimental import pallas as pl
from jax.experimental.pallas import tpu as pltpu

_NCHUNK = 4


def _body(idx_ref, arr_ref, out_ref, buf_ref, in_sems, out_sems):
    i = idx_ref[0]
    c = out_ref.shape[1] // _NCHUNK
    ins = []
    for j in range(_NCHUNK):
        cp = pltpu.make_async_copy(
            arr_ref.at[i, :, pl.ds(j * c, c)],
            buf_ref.at[:, pl.ds(j * c, c)],
            in_sems.at[j],
        )
        cp.start()
        ins.append(cp)
    outs = []
    for j in range(_NCHUNK):
        ins[j].wait()
        cp = pltpu.make_async_copy(
            buf_ref.at[:, pl.ds(j * c, c)],
            out_ref.at[:, pl.ds(j * c, c)],
            out_sems.at[j],
        )
        cp.start()
        outs.append(cp)
    for cp in outs:
        cp.wait()


def kernel(x, arr, index):
    del x
    t, r, d = arr.shape
    idx = jnp.reshape(jnp.asarray(index, jnp.int32), (1,))
    arr_t = jnp.transpose(arr, (0, 2, 1))
    out_t = pl.pallas_call(
        _body,
        out_shape=jax.ShapeDtypeStruct((d, r), jnp.float32),
        in_specs=[
            pl.BlockSpec(memory_space=pltpu.MemorySpace.SMEM),
            pl.BlockSpec(memory_space=pl.ANY),
        ],
        out_specs=pl.BlockSpec(memory_space=pl.ANY),
        scratch_shapes=[
            pltpu.VMEM((d, r), jnp.float32),
            pltpu.SemaphoreType.DMA((_NCHUNK,)),
            pltpu.SemaphoreType.DMA((_NCHUNK,)),
        ],
    )(idx, arr_t)
    return out_t.T
